# baseline (speedup 1.0000x reference)
import jax
import jax.numpy as jnp
from jax.experimental import pallas as pl
from jax.experimental.pallas import tpu as pltpu

_TB = 8


def _vq_body(ze_ref, cb_ref, zq_ref, tok_ref, loss_ref, util_ref):
    step = pl.program_id(0)

    @pl.when(step == 0)
    def _init():
        loss_ref[0, 0] = 0.0
        util_ref[0, 0] = 0.0

    loss_acc = jnp.float32(0.0)
    util_acc = jnp.float32(0.0)
    for s in range(_TB):
        ze = ze_ref[:, s, :]
        cb = cb_ref[s]
        scores = jax.lax.dot_general(
            ze, cb, (((1,), (1,)), ((), ())),
            preferred_element_type=jnp.float32)
        ze_sq = jnp.sum(ze * ze, axis=1, keepdims=True)
        cb_sq = jnp.sum(cb * cb, axis=1)
        dist = ze_sq - 2.0 * scores + cb_sq[None, :]
        mind = jnp.min(dist, axis=1, keepdims=True)
        kio = jax.lax.broadcasted_iota(jnp.int32, dist.shape, 1)
        idx = jnp.min(jnp.where(dist == mind, kio, cb.shape[0]), axis=1)
        onehot = (kio == idx[:, None]).astype(jnp.float32)
        zq = jax.lax.dot_general(
            onehot, cb, (((1,), (0,)), ((), ())),
            preferred_element_type=jnp.float32)
        zq_ref[:, s, :] = zq
        tok_ref[s, :] = idx
        diff = ze - zq
        loss_acc += jnp.sum(diff * diff)
        util_acc += jnp.sum(jnp.max(onehot, axis=0))
    loss_ref[0, 0] += loss_acc
    util_ref[0, 0] += util_acc


def kernel(z_e, codebooks):
    B, T, D = z_e.shape
    K = codebooks.shape[1]
    nsteps = T // _TB
    zq, tok_t, loss, util = pl.pallas_call(
        _vq_body,
        grid=(nsteps,),
        in_specs=[
            pl.BlockSpec((B, _TB, D), lambda i: (0, i, 0)),
            pl.BlockSpec((_TB, K, D), lambda i: (i, 0, 0)),
        ],
        out_specs=[
            pl.BlockSpec((B, _TB, D), lambda i: (0, i, 0)),
            pl.BlockSpec((_TB, B), lambda i: (i, 0)),
            pl.BlockSpec(block_shape=(1, 1), index_map=lambda i: (0, 0),
                         memory_space=pltpu.SMEM),
            pl.BlockSpec(block_shape=(1, 1), index_map=lambda i: (0, 0),
                         memory_space=pltpu.SMEM),
        ],
        out_shape=[
            jax.ShapeDtypeStruct((B, T, D), jnp.float32),
            jax.ShapeDtypeStruct((T, B), jnp.int32),
            jax.ShapeDtypeStruct((1, 1), jnp.float32),
            jax.ShapeDtypeStruct((1, 1), jnp.float32),
        ],
    )(z_e, codebooks)
    vq_loss = 0.25 * loss[0, 0] / (T * B * D)
    utilization = util[0, 0] / (T * K)
    tokens = tok_t.T
    return zq, tokens, vq_loss, utilization
